# baseline (speedup 1.0000x reference)
import functools

import jax
import jax.numpy as jnp
from jax import lax
from jax.experimental import pallas as pl
from jax.experimental.pallas import tpu as pltpu
from jax.experimental.pallas import tpu_sc as plsc

INPUT_DIM = 768
OUTPUT_DIM = 768
NUM_EXPERTS = 16
TOP_K = 2
BATCH = 64

NW = 32
I_PER = INPUT_DIM // NW
L = 16
PK = BATCH * TOP_K
CCH = OUTPUT_DIM // L
BANK_W = NUM_EXPERTS * OUTPUT_DIM
OCH = 32
NOCH = OUTPUT_DIM // OCH
BIAS_W = BATCH // 8


def _iota():
    return lax.broadcasted_iota(jnp.int32, (L,), 0)


def _splat(x):
    return jnp.full((L,), x, jnp.int32)


def _sc_body(wp_hbm, wi_hbm, bp_hbm, bi_hbm, wb_hbm, bb_hbm,
             outw_hbm, outb_hbm,
             bank_a, bank_b, wp_v, wi_v,
             bpc_a, bpc_b, bic_a, bic_b, bbk_a, bbk_b,
             outw_v, outb_v,
             sem_ba, sem_bb, sem_out, sem_bias):
    cid = lax.axis_index("c")
    sid = lax.axis_index("s")
    wid = sid * 2 + cid
    iov = _iota()

    def bank_src(i):
        return wb_hbm.at[pl.ds(i * BANK_W, BANK_W)]

    @pl.when(wid < BIAS_W)
    def _bias():
        b0 = wid * 8

        def fire_bias(ch, bufs):
            off = ch * OCH
            pltpu.async_copy(bp_hbm.at[pl.ds(off * PK, OCH * PK)],
                             bufs[0], sem_bias)
            pltpu.async_copy(bi_hbm.at[pl.ds(off * PK, OCH * PK)],
                             bufs[1], sem_bias)
            pltpu.async_copy(
                bb_hbm.at[pl.ds(off * NUM_EXPERTS, OCH * NUM_EXPERTS)],
                bufs[2], sem_bias)

        def wait_bias(bufs):
            pltpu.make_async_copy(bp_hbm.at[pl.ds(0, OCH * PK)],
                                  bufs[0], sem_bias).wait()
            pltpu.make_async_copy(bi_hbm.at[pl.ds(0, OCH * PK)],
                                  bufs[1], sem_bias).wait()
            pltpu.make_async_copy(
                bb_hbm.at[pl.ds(0, OCH * NUM_EXPERTS)], bufs[2],
                sem_bias).wait()

        bufs = [(bpc_a, bic_a, bbk_a), (bpc_b, bic_b, bbk_b)]
        fire_bias(0, bufs[0])
        for ch in range(NOCH):
            cur = bufs[ch % 2]
            wait_bias(cur)
            if ch + 1 < NOCH:
                fire_bias(ch + 1, bufs[(ch + 1) % 2])
            bp_c, bi_c, bbank_c = cur
            for oc in range(OCH // L):
                olp = (iov + oc * L) * PK
                ole = (iov + oc * L) * NUM_EXPERTS
                for t in range(8):
                    bsp = _splat((b0 + t) * 2)
                    p0 = plsc.load_gather(bp_c, [olp + bsp])
                    p1 = plsc.load_gather(bp_c, [olp + bsp + 1])
                    e0 = plsc.load_gather(bi_c, [olp + bsp])
                    e1 = plsc.load_gather(bi_c, [olp + bsp + 1])
                    v0 = plsc.load_gather(bbank_c, [ole + e0])
                    v1 = plsc.load_gather(bbank_c, [ole + e1])
                    plsc.store_scatter(
                        outb_v,
                        [_splat(t * OUTPUT_DIM + ch * OCH + oc * L) + iov],
                        p0 * v0 + p1 * v1)
        pltpu.sync_copy(outb_v,
                        outb_hbm.at[pl.ds(b0 * OUTPUT_DIM, 8 * OUTPUT_DIM)])

    i0 = wid * I_PER
    pltpu.sync_copy(wp_hbm.at[pl.ds(i0 * PK, I_PER * PK)], wp_v)
    pltpu.sync_copy(wi_hbm.at[pl.ds(i0 * PK, I_PER * PK)], wi_v)

    pltpu.async_copy(bank_src(i0), bank_a, sem_ba)

    def compute_row(i, il, bank_v):
        ilp = _splat(il * PK)

        @plsc.parallel_loop(0, BATCH, step=1, unroll=2)
        def t_body(b):
            bsp = ilp + 2 * b
            p0 = plsc.load_gather(wp_v, [bsp])
            p1 = plsc.load_gather(wp_v, [bsp + 1])
            ec0 = plsc.load_gather(wi_v, [bsp]) * OUTPUT_DIM + iov
            ec1 = plsc.load_gather(wi_v, [bsp + 1]) * OUTPUT_DIM + iov
            ob = _splat(b * OUTPUT_DIM) + iov
            for c in range(CCH):
                v0 = plsc.load_gather(bank_v, [ec0 + c * L])
                v1 = plsc.load_gather(bank_v, [ec1 + c * L])
                plsc.store_scatter(outw_v, [ob + c * L], p0 * v0 + p1 * v1)
            pltpu.async_copy(
                outw_v.at[pl.ds(b * OUTPUT_DIM, OUTPUT_DIM)],
                outw_hbm.at[pl.ds((b * INPUT_DIM + i) * OUTPUT_DIM,
                                  OUTPUT_DIM)],
                sem_out)
        pltpu.make_async_copy(
            outw_v, outw_hbm.at[pl.ds(0, BATCH * OUTPUT_DIM)],
            sem_out).wait()

    def pair_body(p, _):
        i_even = i0 + 2 * p
        pltpu.make_async_copy(bank_src(0), bank_a, sem_ba).wait()
        pltpu.async_copy(bank_src(i_even + 1), bank_b, sem_bb)
        compute_row(i_even, 2 * p, bank_a)
        pltpu.make_async_copy(bank_src(0), bank_b, sem_bb).wait()
        nxt = jnp.minimum(i_even + 2, INPUT_DIM - 1)
        pltpu.async_copy(bank_src(nxt), bank_a, sem_ba)
        compute_row(i_even + 1, 2 * p + 1, bank_b)
        return 0

    lax.fori_loop(0, I_PER // 2, pair_body, 0)
    pltpu.make_async_copy(bank_src(0), bank_a, sem_ba).wait()


@jax.jit
def kernel(weight_probs, weight_indices, bias_probs, bias_indices,
           weight_bank, bias_bank):
    wp = weight_probs.reshape(-1)
    wi = weight_indices.reshape(-1)
    bp = bias_probs.reshape(-1)
    bi = bias_indices.reshape(-1)
    wb = weight_bank.reshape(-1)
    bb = bias_bank.reshape(-1)

    mesh = plsc.VectorSubcoreMesh(core_axis_name="c", subcore_axis_name="s")
    outw, outb = pl.kernel(
        _sc_body,
        out_type=(
            jax.ShapeDtypeStruct((BATCH * INPUT_DIM * OUTPUT_DIM,),
                                 jnp.float32),
            jax.ShapeDtypeStruct((BATCH * OUTPUT_DIM,), jnp.float32),
        ),
        mesh=mesh,
        compiler_params=pltpu.CompilerParams(needs_layout_passes=False),
        scratch_types=(
            pltpu.VMEM((BANK_W,), jnp.float32),
            pltpu.VMEM((BANK_W,), jnp.float32),
            pltpu.VMEM((I_PER * PK,), jnp.float32),
            pltpu.VMEM((I_PER * PK,), jnp.int32),
            pltpu.VMEM((OCH * PK,), jnp.float32),
            pltpu.VMEM((OCH * PK,), jnp.float32),
            pltpu.VMEM((OCH * PK,), jnp.int32),
            pltpu.VMEM((OCH * PK,), jnp.int32),
            pltpu.VMEM((OCH * NUM_EXPERTS,), jnp.float32),
            pltpu.VMEM((OCH * NUM_EXPERTS,), jnp.float32),
            pltpu.VMEM((BATCH * OUTPUT_DIM,), jnp.float32),
            pltpu.VMEM((8 * OUTPUT_DIM,), jnp.float32),
            pltpu.SemaphoreType.DMA,
            pltpu.SemaphoreType.DMA,
            pltpu.SemaphoreType.DMA,
            pltpu.SemaphoreType.DMA,
        ),
    )(wp, wi, bp, bi, wb, bb)
    return (outw.reshape(BATCH, INPUT_DIM, OUTPUT_DIM),
            outb.reshape(BATCH, OUTPUT_DIM))


# speedup vs baseline: 1.0157x; 1.0157x over previous
import functools

import jax
import jax.numpy as jnp
from jax import lax
from jax.experimental import pallas as pl
from jax.experimental.pallas import tpu as pltpu
from jax.experimental.pallas import tpu_sc as plsc

INPUT_DIM = 768
OUTPUT_DIM = 768
NUM_EXPERTS = 16
TOP_K = 2
BATCH = 64

NW = 32
I_PER = INPUT_DIM // NW
L = 16
PK = BATCH * TOP_K
CCH = OUTPUT_DIM // L
BANK_W = NUM_EXPERTS * OUTPUT_DIM
OCH = 32
NOCH = OUTPUT_DIM // OCH
BIAS_W = BATCH // 8


def _iota():
    return lax.broadcasted_iota(jnp.int32, (L,), 0)


def _splat(x):
    return jnp.full((L,), x, jnp.int32)


def _sc_body(wp_hbm, wi_hbm, bp_hbm, bi_hbm, wb_hbm, bb_hbm,
             outw_hbm, outb_hbm,
             bank_a, bank_b, wp_v, wi_v,
             bpc_a, bpc_b, bic_a, bic_b, bbk_a, bbk_b,
             outw_v, outb_v,
             sem_ba, sem_bb, sem_out, sem_bias):
    cid = lax.axis_index("c")
    sid = lax.axis_index("s")
    wid = sid * 2 + cid
    iov = _iota()

    def bank_src(i):
        return wb_hbm.at[pl.ds(i * BANK_W, BANK_W)]

    @pl.when(wid < BIAS_W)
    def _bias():
        b0 = wid * 8

        def fire_bias(ch, bufs):
            off = ch * OCH
            pltpu.async_copy(bp_hbm.at[pl.ds(off * PK, OCH * PK)],
                             bufs[0], sem_bias)
            pltpu.async_copy(bi_hbm.at[pl.ds(off * PK, OCH * PK)],
                             bufs[1], sem_bias)
            pltpu.async_copy(
                bb_hbm.at[pl.ds(off * NUM_EXPERTS, OCH * NUM_EXPERTS)],
                bufs[2], sem_bias)

        def wait_bias(bufs):
            pltpu.make_async_copy(bp_hbm.at[pl.ds(0, OCH * PK)],
                                  bufs[0], sem_bias).wait()
            pltpu.make_async_copy(bi_hbm.at[pl.ds(0, OCH * PK)],
                                  bufs[1], sem_bias).wait()
            pltpu.make_async_copy(
                bb_hbm.at[pl.ds(0, OCH * NUM_EXPERTS)], bufs[2],
                sem_bias).wait()

        bufs = [(bpc_a, bic_a, bbk_a), (bpc_b, bic_b, bbk_b)]
        fire_bias(0, bufs[0])
        for ch in range(NOCH):
            cur = bufs[ch % 2]
            wait_bias(cur)
            if ch + 1 < NOCH:
                fire_bias(ch + 1, bufs[(ch + 1) % 2])
            bp_c, bi_c, bbank_c = cur
            for oc in range(OCH // L):
                olp = (iov + oc * L) * PK
                ole = (iov + oc * L) * NUM_EXPERTS
                for t in range(8):
                    bsp = _splat((b0 + t) * 2)
                    p0 = plsc.load_gather(bp_c, [olp + bsp])
                    p1 = plsc.load_gather(bp_c, [olp + bsp + 1])
                    e0 = plsc.load_gather(bi_c, [olp + bsp])
                    e1 = plsc.load_gather(bi_c, [olp + bsp + 1])
                    v0 = plsc.load_gather(bbank_c, [ole + e0])
                    v1 = plsc.load_gather(bbank_c, [ole + e1])
                    plsc.store_scatter(
                        outb_v,
                        [_splat(t * OUTPUT_DIM + ch * OCH + oc * L) + iov],
                        p0 * v0 + p1 * v1)
        pltpu.sync_copy(outb_v,
                        outb_hbm.at[pl.ds(b0 * OUTPUT_DIM, 8 * OUTPUT_DIM)])

    i0 = wid * I_PER
    pltpu.sync_copy(wp_hbm.at[pl.ds(i0 * PK, I_PER * PK)], wp_v)
    pltpu.sync_copy(wi_hbm.at[pl.ds(i0 * PK, I_PER * PK)], wi_v)

    pltpu.async_copy(bank_src(i0), bank_a, sem_ba)

    def compute_row(i, il, bank_v):
        ilp = _splat(il * PK)

        @plsc.parallel_loop(0, BATCH, step=1, unroll=2)
        def t_body(b):
            bsp = ilp + 2 * b
            p0 = plsc.load_gather(wp_v, [bsp])
            p1 = plsc.load_gather(wp_v, [bsp + 1])
            ec0 = plsc.load_gather(wi_v, [bsp]) * OUTPUT_DIM + iov
            ec1 = plsc.load_gather(wi_v, [bsp + 1]) * OUTPUT_DIM + iov
            ob = b * OUTPUT_DIM
            for c in range(CCH):
                v0 = plsc.load_gather(bank_v, [ec0 + c * L])
                v1 = plsc.load_gather(bank_v, [ec1 + c * L])
                outw_v[pl.ds(ob + c * L, L)] = p0 * v0 + p1 * v1
            pltpu.async_copy(
                outw_v.at[pl.ds(b * OUTPUT_DIM, OUTPUT_DIM)],
                outw_hbm.at[pl.ds((b * INPUT_DIM + i) * OUTPUT_DIM,
                                  OUTPUT_DIM)],
                sem_out)
        pltpu.make_async_copy(
            outw_v, outw_hbm.at[pl.ds(0, BATCH * OUTPUT_DIM)],
            sem_out).wait()

    def pair_body(p, _):
        i_even = i0 + 2 * p
        pltpu.make_async_copy(bank_src(0), bank_a, sem_ba).wait()
        pltpu.async_copy(bank_src(i_even + 1), bank_b, sem_bb)
        compute_row(i_even, 2 * p, bank_a)
        pltpu.make_async_copy(bank_src(0), bank_b, sem_bb).wait()
        nxt = jnp.minimum(i_even + 2, INPUT_DIM - 1)
        pltpu.async_copy(bank_src(nxt), bank_a, sem_ba)
        compute_row(i_even + 1, 2 * p + 1, bank_b)
        return 0

    lax.fori_loop(0, I_PER // 2, pair_body, 0)
    pltpu.make_async_copy(bank_src(0), bank_a, sem_ba).wait()


@jax.jit
def kernel(weight_probs, weight_indices, bias_probs, bias_indices,
           weight_bank, bias_bank):
    wp = weight_probs.reshape(-1)
    wi = weight_indices.reshape(-1)
    bp = bias_probs.reshape(-1)
    bi = bias_indices.reshape(-1)
    wb = weight_bank.reshape(-1)
    bb = bias_bank.reshape(-1)

    mesh = plsc.VectorSubcoreMesh(core_axis_name="c", subcore_axis_name="s")
    outw, outb = pl.kernel(
        _sc_body,
        out_type=(
            jax.ShapeDtypeStruct((BATCH * INPUT_DIM * OUTPUT_DIM,),
                                 jnp.float32),
            jax.ShapeDtypeStruct((BATCH * OUTPUT_DIM,), jnp.float32),
        ),
        mesh=mesh,
        compiler_params=pltpu.CompilerParams(needs_layout_passes=False),
        scratch_types=(
            pltpu.VMEM((BANK_W,), jnp.float32),
            pltpu.VMEM((BANK_W,), jnp.float32),
            pltpu.VMEM((I_PER * PK,), jnp.float32),
            pltpu.VMEM((I_PER * PK,), jnp.int32),
            pltpu.VMEM((OCH * PK,), jnp.float32),
            pltpu.VMEM((OCH * PK,), jnp.float32),
            pltpu.VMEM((OCH * PK,), jnp.int32),
            pltpu.VMEM((OCH * PK,), jnp.int32),
            pltpu.VMEM((OCH * NUM_EXPERTS,), jnp.float32),
            pltpu.VMEM((OCH * NUM_EXPERTS,), jnp.float32),
            pltpu.VMEM((BATCH * OUTPUT_DIM,), jnp.float32),
            pltpu.VMEM((8 * OUTPUT_DIM,), jnp.float32),
            pltpu.SemaphoreType.DMA,
            pltpu.SemaphoreType.DMA,
            pltpu.SemaphoreType.DMA,
            pltpu.SemaphoreType.DMA,
        ),
    )(wp, wi, bp, bi, wb, bb)
    return (outw.reshape(BATCH, INPUT_DIM, OUTPUT_DIM),
            outb.reshape(BATCH, OUTPUT_DIM))


# speedup vs baseline: 1.7958x; 1.7681x over previous
import functools

import jax
import jax.numpy as jnp
from jax import lax
from jax.experimental import pallas as pl
from jax.experimental.pallas import tpu as pltpu
from jax.experimental.pallas import tpu_sc as plsc

INPUT_DIM = 768
OUTPUT_DIM = 768
NUM_EXPERTS = 16
TOP_K = 2
BATCH = 64

NW = 32
I_PER = INPUT_DIM // NW
L = 16
PK = BATCH * TOP_K
CCH = OUTPUT_DIM // L
BANK_W = NUM_EXPERTS * OUTPUT_DIM
OCH = 32
NOCH = OUTPUT_DIM // OCH
BIAS_W = BATCH // 8


def _iota():
    return lax.broadcasted_iota(jnp.int32, (L,), 0)


def _splat(x):
    return jnp.full((L,), x, jnp.int32)


def _sc_body(wp_hbm, wi_hbm, bp_hbm, bi_hbm, wb_hbm, bb_hbm,
             outw_hbm, outb_hbm,
             bank_a, bank_b, wp_v, wi_v,
             bpc_a, bpc_b, bic_a, bic_b, bbk_a, bbk_b,
             outw_a, outw_b, idx_a, idx_b, outb_v,
             sem_ba, sem_bb, sem_out, sem_bias):
    cid = lax.axis_index("c")
    sid = lax.axis_index("s")
    wid = sid * 2 + cid
    iov = _iota()

    def bank_src(i):
        return wb_hbm.at[pl.ds(i * BANK_W, BANK_W)]

    @pl.when(wid < BIAS_W)
    def _bias():
        b0 = wid * 8

        def fire_bias(ch, bufs):
            off = ch * OCH
            pltpu.async_copy(bp_hbm.at[pl.ds(off * PK, OCH * PK)],
                             bufs[0], sem_bias)
            pltpu.async_copy(bi_hbm.at[pl.ds(off * PK, OCH * PK)],
                             bufs[1], sem_bias)
            pltpu.async_copy(
                bb_hbm.at[pl.ds(off * NUM_EXPERTS, OCH * NUM_EXPERTS)],
                bufs[2], sem_bias)

        def wait_bias(bufs):
            pltpu.make_async_copy(bp_hbm.at[pl.ds(0, OCH * PK)],
                                  bufs[0], sem_bias).wait()
            pltpu.make_async_copy(bi_hbm.at[pl.ds(0, OCH * PK)],
                                  bufs[1], sem_bias).wait()
            pltpu.make_async_copy(
                bb_hbm.at[pl.ds(0, OCH * NUM_EXPERTS)], bufs[2],
                sem_bias).wait()

        bufs = [(bpc_a, bic_a, bbk_a), (bpc_b, bic_b, bbk_b)]
        fire_bias(0, bufs[0])
        for ch in range(NOCH):
            cur = bufs[ch % 2]
            wait_bias(cur)
            if ch + 1 < NOCH:
                fire_bias(ch + 1, bufs[(ch + 1) % 2])
            bp_c, bi_c, bbank_c = cur
            for oc in range(OCH // L):
                olp = (iov + oc * L) * PK
                ole = (iov + oc * L) * NUM_EXPERTS
                for t in range(8):
                    bsp = _splat((b0 + t) * 2)
                    p0 = plsc.load_gather(bp_c, [olp + bsp])
                    p1 = plsc.load_gather(bp_c, [olp + bsp + 1])
                    e0 = plsc.load_gather(bi_c, [olp + bsp])
                    e1 = plsc.load_gather(bi_c, [olp + bsp + 1])
                    v0 = plsc.load_gather(bbank_c, [ole + e0])
                    v1 = plsc.load_gather(bbank_c, [ole + e1])
                    plsc.store_scatter(
                        outb_v,
                        [_splat(t * OUTPUT_DIM + ch * OCH + oc * L) + iov],
                        p0 * v0 + p1 * v1)
        pltpu.sync_copy(outb_v,
                        outb_hbm.at[pl.ds(b0 * OUTPUT_DIM, 8 * OUTPUT_DIM)])

    i0 = wid * I_PER
    pltpu.sync_copy(wp_hbm.at[pl.ds(i0 * PK, I_PER * PK)], wp_v)
    pltpu.sync_copy(wi_hbm.at[pl.ds(i0 * PK, I_PER * PK)], wi_v)

    pltpu.async_copy(bank_src(i0), bank_a, sem_ba)

    def compute_row(i, il, bank_v):
        ilp = _splat(il * PK)

        def half_step(half, buf, idx_v):
            @pl.when(il * 2 + half >= 2)
            def _():
                pltpu.make_async_copy(buf, outw_hbm.at[idx_v],
                                      sem_out).wait()

            @plsc.parallel_loop(0, BATCH // 2, step=1, unroll=2)
            def t_body(t):
                b = half * (BATCH // 2) + t
                bsp = ilp + 2 * b
                p0 = plsc.load_gather(wp_v, [bsp])
                p1 = plsc.load_gather(wp_v, [bsp + 1])
                ec0 = plsc.load_gather(wi_v, [bsp]) * OUTPUT_DIM + iov
                ec1 = plsc.load_gather(wi_v, [bsp + 1]) * OUTPUT_DIM + iov
                for c in range(CCH):
                    v0 = plsc.load_gather(bank_v, [ec0 + c * L])
                    v1 = plsc.load_gather(bank_v, [ec1 + c * L])
                    buf[t, pl.ds(c * L, L)] = p0 * v0 + p1 * v1

            for tg in range(2):
                idx_v[pl.ds(tg * L, L)] = (
                    (iov + half * (BATCH // 2) + tg * L) * INPUT_DIM + i)
            pltpu.async_copy(buf, outw_hbm.at[idx_v], sem_out)

        half_step(0, outw_a, idx_a)
        half_step(1, outw_b, idx_b)

    def pair_body(p, _):
        i_even = i0 + 2 * p
        pltpu.make_async_copy(bank_src(0), bank_a, sem_ba).wait()
        pltpu.async_copy(bank_src(i_even + 1), bank_b, sem_bb)
        compute_row(i_even, 2 * p, bank_a)
        pltpu.make_async_copy(bank_src(0), bank_b, sem_bb).wait()
        nxt = jnp.minimum(i_even + 2, INPUT_DIM - 1)
        pltpu.async_copy(bank_src(nxt), bank_a, sem_ba)
        compute_row(i_even + 1, 2 * p + 1, bank_b)
        return 0

    lax.fori_loop(0, I_PER // 2, pair_body, 0)
    pltpu.make_async_copy(bank_src(0), bank_a, sem_ba).wait()
    pltpu.make_async_copy(outw_a, outw_hbm.at[idx_a], sem_out).wait()
    pltpu.make_async_copy(outw_b, outw_hbm.at[idx_b], sem_out).wait()


@jax.jit
def kernel(weight_probs, weight_indices, bias_probs, bias_indices,
           weight_bank, bias_bank):
    wp = weight_probs.reshape(-1)
    wi = weight_indices.reshape(-1)
    bp = bias_probs.reshape(-1)
    bi = bias_indices.reshape(-1)
    wb = weight_bank.reshape(-1)
    bb = bias_bank.reshape(-1)

    mesh = plsc.VectorSubcoreMesh(core_axis_name="c", subcore_axis_name="s")
    outw, outb = pl.kernel(
        _sc_body,
        out_type=(
            jax.ShapeDtypeStruct((BATCH * INPUT_DIM, OUTPUT_DIM),
                                 jnp.float32),
            jax.ShapeDtypeStruct((BATCH * OUTPUT_DIM,), jnp.float32),
        ),
        mesh=mesh,
        compiler_params=pltpu.CompilerParams(needs_layout_passes=False),
        scratch_types=(
            pltpu.VMEM((BANK_W,), jnp.float32),
            pltpu.VMEM((BANK_W,), jnp.float32),
            pltpu.VMEM((I_PER * PK,), jnp.float32),
            pltpu.VMEM((I_PER * PK,), jnp.int32),
            pltpu.VMEM((OCH * PK,), jnp.float32),
            pltpu.VMEM((OCH * PK,), jnp.float32),
            pltpu.VMEM((OCH * PK,), jnp.int32),
            pltpu.VMEM((OCH * PK,), jnp.int32),
            pltpu.VMEM((OCH * NUM_EXPERTS,), jnp.float32),
            pltpu.VMEM((OCH * NUM_EXPERTS,), jnp.float32),
            pltpu.VMEM((BATCH // 2, OUTPUT_DIM), jnp.float32),
            pltpu.VMEM((BATCH // 2, OUTPUT_DIM), jnp.float32),
            pltpu.VMEM((BATCH // 2,), jnp.int32),
            pltpu.VMEM((BATCH // 2,), jnp.int32),
            pltpu.VMEM((8 * OUTPUT_DIM,), jnp.float32),
            pltpu.SemaphoreType.DMA,
            pltpu.SemaphoreType.DMA,
            pltpu.SemaphoreType.DMA,
            pltpu.SemaphoreType.DMA,
        ),
    )(wp, wi, bp, bi, wb, bb)
    return (outw.reshape(BATCH, INPUT_DIM, OUTPUT_DIM),
            outb.reshape(BATCH, OUTPUT_DIM))


# speedup vs baseline: 2.8434x; 1.5833x over previous
import functools

import jax
import jax.numpy as jnp
from jax import lax
from jax.experimental import pallas as pl
from jax.experimental.pallas import tpu as pltpu
from jax.experimental.pallas import tpu_sc as plsc

INPUT_DIM = 768
OUTPUT_DIM = 768
NUM_EXPERTS = 16
TOP_K = 2
BATCH = 64

NW = 32
I_PER = INPUT_DIM // NW
L = 16
PK = BATCH * TOP_K
ROW_W = BATCH * NUM_EXPERTS
S_W = I_PER * ROW_W
OCH = 32
NOCH = OUTPUT_DIM // OCH
BIAS_W = BATCH // 8

BI = 16


def _iota():
    return lax.broadcasted_iota(jnp.int32, (L,), 0)


def _splat(x):
    return jnp.full((L,), x, jnp.int32)


def _sc_body(wp_hbm, wi_hbm, bp_hbm, bi_hbm, bb_hbm,
             s_hbm, outb_hbm,
             wp_v, wi_v, s_v,
             bpc_a, bpc_b, bic_a, bic_b, bbk_a, bbk_b,
             outb_v, sem_bias):
    cid = lax.axis_index("c")
    sid = lax.axis_index("s")
    wid = sid * 2 + cid
    iov = _iota()

    i0 = wid * I_PER
    pltpu.sync_copy(wp_hbm.at[pl.ds(i0 * PK, I_PER * PK)], wp_v)
    pltpu.sync_copy(wi_hbm.at[pl.ds(i0 * PK, I_PER * PK)], wi_v)

    zv = jnp.zeros((L,), jnp.float32)

    @plsc.parallel_loop(0, S_W // L, step=1, unroll=8)
    def zero_body(j):
        plsc.store_scatter(s_v, [_splat(j * L) + iov], zv)

    def row_body(il, _):
        for g in range(BATCH // L):
            src = _splat(il * PK + g * 2 * L) + iov * 2
            base = _splat(il * ROW_W + g * L * NUM_EXPERTS)
            p0 = plsc.load_gather(wp_v, [src])
            e0 = plsc.load_gather(wi_v, [src])
            t0 = base + iov * NUM_EXPERTS + e0
            plsc.store_scatter(s_v, [t0], p0)
            p1 = plsc.load_gather(wp_v, [src + 1])
            e1 = plsc.load_gather(wi_v, [src + 1])
            t1 = base + iov * NUM_EXPERTS + e1
            cur = plsc.load_gather(s_v, [t1])
            plsc.store_scatter(s_v, [t1], cur + p1)
        return 0

    lax.fori_loop(0, I_PER, row_body, 0)
    pltpu.sync_copy(s_v, s_hbm.at[pl.ds(wid * S_W, S_W)])

    @pl.when(wid < BIAS_W)
    def _bias():
        b0 = wid * 8

        def fire_bias(ch, bufs):
            off = ch * OCH
            pltpu.async_copy(bp_hbm.at[pl.ds(off * PK, OCH * PK)],
                             bufs[0], sem_bias)
            pltpu.async_copy(bi_hbm.at[pl.ds(off * PK, OCH * PK)],
                             bufs[1], sem_bias)
            pltpu.async_copy(
                bb_hbm.at[pl.ds(off * NUM_EXPERTS, OCH * NUM_EXPERTS)],
                bufs[2], sem_bias)

        def wait_bias(bufs):
            pltpu.make_async_copy(bp_hbm.at[pl.ds(0, OCH * PK)],
                                  bufs[0], sem_bias).wait()
            pltpu.make_async_copy(bi_hbm.at[pl.ds(0, OCH * PK)],
                                  bufs[1], sem_bias).wait()
            pltpu.make_async_copy(
                bb_hbm.at[pl.ds(0, OCH * NUM_EXPERTS)], bufs[2],
                sem_bias).wait()

        bufs = [(bpc_a, bic_a, bbk_a), (bpc_b, bic_b, bbk_b)]
        fire_bias(0, bufs[0])
        for ch in range(NOCH):
            cur = bufs[ch % 2]
            wait_bias(cur)
            if ch + 1 < NOCH:
                fire_bias(ch + 1, bufs[(ch + 1) % 2])
            bp_c, bi_c, bbank_c = cur
            for oc in range(OCH // L):
                olp = (iov + oc * L) * PK
                ole = (iov + oc * L) * NUM_EXPERTS
                for t in range(8):
                    bsp = _splat((b0 + t) * 2)
                    p0 = plsc.load_gather(bp_c, [olp + bsp])
                    p1 = plsc.load_gather(bp_c, [olp + bsp + 1])
                    e0 = plsc.load_gather(bi_c, [olp + bsp])
                    e1 = plsc.load_gather(bi_c, [olp + bsp + 1])
                    v0 = plsc.load_gather(bbank_c, [ole + e0])
                    v1 = plsc.load_gather(bbank_c, [ole + e1])
                    plsc.store_scatter(
                        outb_v,
                        [_splat(t * OUTPUT_DIM + ch * OCH + oc * L) + iov],
                        p0 * v0 + p1 * v1)
        pltpu.sync_copy(outb_v,
                        outb_hbm.at[pl.ds(b0 * OUTPUT_DIM, 8 * OUTPUT_DIM)])


def _weight_body(s_ref, bank_ref, out_ref):
    res = jax.lax.dot_general(
        s_ref[...], bank_ref[...],
        dimension_numbers=(((2,), (1,)), ((0,), (0,))),
        preferred_element_type=jnp.float32)
    out_ref[...] = jnp.transpose(res, (1, 0, 2))


@jax.jit
def kernel(weight_probs, weight_indices, bias_probs, bias_indices,
           weight_bank, bias_bank):
    wp = weight_probs.reshape(-1)
    wi = weight_indices.reshape(-1)
    bp = bias_probs.reshape(-1)
    bi = bias_indices.reshape(-1)
    bb = bias_bank.reshape(-1)

    mesh = plsc.VectorSubcoreMesh(core_axis_name="c", subcore_axis_name="s")
    s_flat, outb = pl.kernel(
        _sc_body,
        out_type=(
            jax.ShapeDtypeStruct((INPUT_DIM * BATCH * NUM_EXPERTS,),
                                 jnp.float32),
            jax.ShapeDtypeStruct((BATCH * OUTPUT_DIM,), jnp.float32),
        ),
        mesh=mesh,
        compiler_params=pltpu.CompilerParams(needs_layout_passes=False),
        scratch_types=(
            pltpu.VMEM((I_PER * PK,), jnp.float32),
            pltpu.VMEM((I_PER * PK,), jnp.int32),
            pltpu.VMEM((S_W,), jnp.float32),
            pltpu.VMEM((OCH * PK,), jnp.float32),
            pltpu.VMEM((OCH * PK,), jnp.float32),
            pltpu.VMEM((OCH * PK,), jnp.int32),
            pltpu.VMEM((OCH * PK,), jnp.int32),
            pltpu.VMEM((OCH * NUM_EXPERTS,), jnp.float32),
            pltpu.VMEM((OCH * NUM_EXPERTS,), jnp.float32),
            pltpu.VMEM((8 * OUTPUT_DIM,), jnp.float32),
            pltpu.SemaphoreType.DMA,
        ),
    )(wp, wi, bp, bi, bb)

    s = s_flat.reshape(INPUT_DIM, BATCH, NUM_EXPERTS)
    nblk = INPUT_DIM // BI
    weight_mixture = pl.pallas_call(
        _weight_body,
        grid=(nblk,),
        in_specs=[
            pl.BlockSpec((BI, BATCH, NUM_EXPERTS), lambda i: (i, 0, 0)),
            pl.BlockSpec((BI, NUM_EXPERTS, OUTPUT_DIM), lambda i: (i, 0, 0)),
        ],
        out_specs=pl.BlockSpec((BATCH, BI, OUTPUT_DIM), lambda i: (0, i, 0)),
        out_shape=jax.ShapeDtypeStruct((BATCH, INPUT_DIM, OUTPUT_DIM),
                                       jnp.float32),
    )(s, weight_bank)

    return weight_mixture, outb.reshape(BATCH, OUTPUT_DIM)


# speedup vs baseline: 4.7493x; 1.6703x over previous
import functools

import jax
import jax.numpy as jnp
from jax import lax
from jax.experimental import pallas as pl
from jax.experimental.pallas import tpu as pltpu
from jax.experimental.pallas import tpu_sc as plsc

INPUT_DIM = 768
OUTPUT_DIM = 768
NUM_EXPERTS = 16
TOP_K = 2
BATCH = 64

L = 16
PK = BATCH * TOP_K
OCH = 32
NOCH = OUTPUT_DIM // OCH
BIAS_W = BATCH // 8

BI = 16


def _iota():
    return lax.broadcasted_iota(jnp.int32, (L,), 0)


def _splat(x):
    return jnp.full((L,), x, jnp.int32)


def _sc_body(bp_hbm, bi_hbm, bb_hbm, outb_hbm,
             bpc_a, bpc_b, bic_a, bic_b, bbk_a, bbk_b,
             outb_v, sem_bias):
    cid = lax.axis_index("c")
    sid = lax.axis_index("s")
    wid = sid * 2 + cid
    iov = _iota()

    @pl.when(wid < BIAS_W)
    def _bias():
        b0 = wid * 8

        def fire_bias(ch, bufs):
            off = ch * OCH
            pltpu.async_copy(bp_hbm.at[pl.ds(off * PK, OCH * PK)],
                             bufs[0], sem_bias)
            pltpu.async_copy(bi_hbm.at[pl.ds(off * PK, OCH * PK)],
                             bufs[1], sem_bias)
            pltpu.async_copy(
                bb_hbm.at[pl.ds(off * NUM_EXPERTS, OCH * NUM_EXPERTS)],
                bufs[2], sem_bias)

        def wait_bias(bufs):
            pltpu.make_async_copy(bp_hbm.at[pl.ds(0, OCH * PK)],
                                  bufs[0], sem_bias).wait()
            pltpu.make_async_copy(bi_hbm.at[pl.ds(0, OCH * PK)],
                                  bufs[1], sem_bias).wait()
            pltpu.make_async_copy(
                bb_hbm.at[pl.ds(0, OCH * NUM_EXPERTS)], bufs[2],
                sem_bias).wait()

        bufs = [(bpc_a, bic_a, bbk_a), (bpc_b, bic_b, bbk_b)]
        fire_bias(0, bufs[0])
        for ch in range(NOCH):
            cur = bufs[ch % 2]
            wait_bias(cur)
            if ch + 1 < NOCH:
                fire_bias(ch + 1, bufs[(ch + 1) % 2])
            bp_c, bi_c, bbank_c = cur
            for oc in range(OCH // L):
                olp = (iov + oc * L) * PK
                ole = (iov + oc * L) * NUM_EXPERTS
                for t in range(8):
                    bsp = _splat((b0 + t) * 2)
                    p0 = plsc.load_gather(bp_c, [olp + bsp])
                    p1 = plsc.load_gather(bp_c, [olp + bsp + 1])
                    e0 = plsc.load_gather(bi_c, [olp + bsp])
                    e1 = plsc.load_gather(bi_c, [olp + bsp + 1])
                    v0 = plsc.load_gather(bbank_c, [ole + e0])
                    v1 = plsc.load_gather(bbank_c, [ole + e1])
                    plsc.store_scatter(
                        outb_v,
                        [_splat(t * OUTPUT_DIM + ch * OCH + oc * L) + iov],
                        p0 * v0 + p1 * v1)
        pltpu.sync_copy(outb_v,
                        outb_hbm.at[pl.ds(b0 * OUTPUT_DIM, 8 * OUTPUT_DIM)])


def _weight_body(wp0_ref, wp1_ref, wi0_ref, wi1_ref, bank_ref, out_ref):
    bank = bank_ref[...]
    e_iota = lax.broadcasted_iota(jnp.int32, (BI, BATCH, NUM_EXPERTS), 2)
    s = jnp.where(wi0_ref[...][:, :, None] == e_iota,
                  wp0_ref[...][:, :, None], 0.0)
    s = s + jnp.where(wi1_ref[...][:, :, None] == e_iota,
                      wp1_ref[...][:, :, None], 0.0)
    res = lax.dot_general(
        s, bank,
        dimension_numbers=(((2,), (1,)), ((0,), (0,))),
        preferred_element_type=jnp.float32)
    out_ref[...] = jnp.transpose(res, (1, 0, 2))


@jax.jit
def kernel(weight_probs, weight_indices, bias_probs, bias_indices,
           weight_bank, bias_bank):
    bp = bias_probs.reshape(-1)
    bi = bias_indices.reshape(-1)
    bb = bias_bank.reshape(-1)

    mesh = plsc.VectorSubcoreMesh(core_axis_name="c", subcore_axis_name="s")
    bias_mixture = pl.kernel(
        _sc_body,
        out_type=jax.ShapeDtypeStruct((BATCH * OUTPUT_DIM,), jnp.float32),
        mesh=mesh,
        compiler_params=pltpu.CompilerParams(needs_layout_passes=False),
        scratch_types=(
            pltpu.VMEM((OCH * PK,), jnp.float32),
            pltpu.VMEM((OCH * PK,), jnp.float32),
            pltpu.VMEM((OCH * PK,), jnp.int32),
            pltpu.VMEM((OCH * PK,), jnp.int32),
            pltpu.VMEM((OCH * NUM_EXPERTS,), jnp.float32),
            pltpu.VMEM((OCH * NUM_EXPERTS,), jnp.float32),
            pltpu.VMEM((8 * OUTPUT_DIM,), jnp.float32),
            pltpu.SemaphoreType.DMA,
        ),
    )(bp, bi, bb)

    wp0, wp1 = weight_probs[:, :, 0], weight_probs[:, :, 1]
    wi0, wi1 = weight_indices[:, :, 0], weight_indices[:, :, 1]
    nblk = INPUT_DIM // BI
    dxb = pl.BlockSpec((BI, BATCH), lambda i: (i, 0))
    weight_mixture = pl.pallas_call(
        _weight_body,
        grid=(nblk,),
        in_specs=[
            dxb, dxb, dxb, dxb,
            pl.BlockSpec((BI, NUM_EXPERTS, OUTPUT_DIM), lambda i: (i, 0, 0)),
        ],
        out_specs=pl.BlockSpec((BATCH, BI, OUTPUT_DIM), lambda i: (0, i, 0)),
        out_shape=jax.ShapeDtypeStruct((BATCH, INPUT_DIM, OUTPUT_DIM),
                                       jnp.float32),
    )(wp0, wp1, wi0, wi1, weight_bank)

    return weight_mixture, bias_mixture.reshape(BATCH, OUTPUT_DIM)


# speedup vs baseline: 4.7909x; 1.0088x over previous
import functools

import jax
import jax.numpy as jnp
from jax import lax
from jax.experimental import pallas as pl
from jax.experimental.pallas import tpu as pltpu
from jax.experimental.pallas import tpu_sc as plsc

INPUT_DIM = 768
OUTPUT_DIM = 768
NUM_EXPERTS = 16
TOP_K = 2
BATCH = 64

L = 16
PK = BATCH * TOP_K
NW = 32
R_PER = OUTPUT_DIM // NW

BI = 16


def _iota():
    return lax.broadcasted_iota(jnp.int32, (L,), 0)


def _splat(x):
    return jnp.full((L,), x, jnp.int32)


def _sc_body(bp_hbm, bi_hbm, bb_hbm, outbT_hbm,
             bp_v, bi_v, bb_v, outb_v):
    cid = lax.axis_index("c")
    sid = lax.axis_index("s")
    wid = sid * 2 + cid
    iov = _iota()
    r0 = wid * R_PER

    pltpu.sync_copy(bp_hbm.at[pl.ds(r0 * PK, R_PER * PK)], bp_v)
    pltpu.sync_copy(bi_hbm.at[pl.ds(r0 * PK, R_PER * PK)], bi_v)
    pltpu.sync_copy(bb_hbm.at[pl.ds(r0 * NUM_EXPERTS, R_PER * NUM_EXPERTS)],
                    bb_v)

    for rl in range(R_PER):
        ebase = _splat(rl * NUM_EXPERTS)
        for g in range(BATCH // L):
            src = _splat(rl * PK + g * 2 * L) + iov * 2
            p0 = plsc.load_gather(bp_v, [src])
            e0 = plsc.load_gather(bi_v, [src])
            p1 = plsc.load_gather(bp_v, [src + 1])
            e1 = plsc.load_gather(bi_v, [src + 1])
            v0 = plsc.load_gather(bb_v, [ebase + e0])
            v1 = plsc.load_gather(bb_v, [ebase + e1])
            outb_v[rl, pl.ds(g * L, L)] = p0 * v0 + p1 * v1

    pltpu.sync_copy(outb_v, outbT_hbm.at[pl.ds(r0, R_PER)])


def _weight_body(wp0_ref, wp1_ref, wi0_ref, wi1_ref, bank_ref, out_ref):
    bank = bank_ref[...]
    e_iota = lax.broadcasted_iota(jnp.int32, (BI, BATCH, NUM_EXPERTS), 2)
    s = jnp.where(wi0_ref[...][:, :, None] == e_iota,
                  wp0_ref[...][:, :, None], 0.0)
    s = s + jnp.where(wi1_ref[...][:, :, None] == e_iota,
                      wp1_ref[...][:, :, None], 0.0)
    res = lax.dot_general(
        s, bank,
        dimension_numbers=(((2,), (1,)), ((0,), (0,))),
        preferred_element_type=jnp.float32)
    out_ref[...] = jnp.transpose(res, (1, 0, 2))


@jax.jit
def kernel(weight_probs, weight_indices, bias_probs, bias_indices,
           weight_bank, bias_bank):
    bp = bias_probs.reshape(-1)
    bi = bias_indices.reshape(-1)
    bb = bias_bank.reshape(-1)

    mesh = plsc.VectorSubcoreMesh(core_axis_name="c", subcore_axis_name="s")
    outbT = pl.kernel(
        _sc_body,
        out_type=jax.ShapeDtypeStruct((OUTPUT_DIM, BATCH), jnp.float32),
        mesh=mesh,
        compiler_params=pltpu.CompilerParams(needs_layout_passes=False),
        scratch_types=(
            pltpu.VMEM((R_PER * PK,), jnp.float32),
            pltpu.VMEM((R_PER * PK,), jnp.int32),
            pltpu.VMEM((R_PER * NUM_EXPERTS,), jnp.float32),
            pltpu.VMEM((R_PER, BATCH), jnp.float32),
        ),
    )(bp, bi, bb)

    wp0, wp1 = weight_probs[:, :, 0], weight_probs[:, :, 1]
    wi0, wi1 = weight_indices[:, :, 0], weight_indices[:, :, 1]
    nblk = INPUT_DIM // BI
    dxb = pl.BlockSpec((BI, BATCH), lambda i: (i, 0))
    weight_mixture = pl.pallas_call(
        _weight_body,
        grid=(nblk,),
        in_specs=[
            dxb, dxb, dxb, dxb,
            pl.BlockSpec((BI, NUM_EXPERTS, OUTPUT_DIM), lambda i: (i, 0, 0)),
        ],
        out_specs=pl.BlockSpec((BATCH, BI, OUTPUT_DIM), lambda i: (0, i, 0)),
        out_shape=jax.ShapeDtypeStruct((BATCH, INPUT_DIM, OUTPUT_DIM),
                                       jnp.float32),
    )(wp0, wp1, wi0, wi1, weight_bank)

    return weight_mixture, outbT.T
